# Initial kernel scaffold; baseline (speedup 1.0000x reference)
import jax
import jax.numpy as jnp
from jax.experimental import pallas as pl


def kernel(x, edge_index, edge_index_sim, batch, W1, b1, Wg, bg, Ws, bs, Ww, bw):
    raise NotImplementedError("write your pallas kernel here")


# speedup vs baseline: 6.5218x; 6.5218x over previous
import functools

import jax
import jax.numpy as jnp
from jax import lax
from jax.experimental import pallas as pl
from jax.experimental.pallas import tpu as pltpu
from jax.experimental.pallas import tpu_sc as plsc

H = 64
HH = 32
G = 64
NSUB = 16
C = 80
ZR = 392
NPAD = 16 * ZR * 8


def _make_deg(N, E):
    EP = E // NSUB
    NCH = EP // C
    RP = N // NSUB
    NZ = RP // ZR
    mesh = plsc.VectorSubcoreMesh(core_axis_name="c", subcore_axis_name="s")
    out_t = jax.ShapeDtypeStruct((2 * N, 16), jnp.float32)

    @functools.partial(
        pl.kernel, out_type=out_t, mesh=mesh,
        compiler_params=pltpu.CompilerParams(use_tc_tiling_on_sc=False),
        scratch_types=[
            pltpu.VMEM((C,), jnp.int32),
            pltpu.VMEM((C, 16), jnp.float32),
            pltpu.VMEM_SHARED((N, 16), jnp.float32),
        ],
    )
    def deg(dstcat_h, ones_h, z16_h, out, dst_v, obuf, acc):
        cid = lax.axis_index("c")
        sid = lax.axis_index("s")
        pltpu.sync_copy(ones_h, obuf)

        def zb(i, carry):
            pltpu.sync_copy(z16_h, acc.at[pl.ds(sid * RP + i * ZR, ZR)])
            return carry
        lax.fori_loop(0, NZ, zb, 0)
        plsc.subcore_barrier()

        def ch(j, carry):
            b = cid * E + sid * EP + j * C
            pltpu.sync_copy(dstcat_h.at[pl.ds(b, C)], dst_v)
            pltpu.sync_copy(obuf, acc.at[dst_v], add=True)
            return carry
        lax.fori_loop(0, NCH, ch, 0)
        plsc.subcore_barrier()

        rb = sid * RP
        pltpu.sync_copy(acc.at[pl.ds(rb, RP)], out.at[pl.ds(cid * N + rb, RP)])

    return deg


def _make_prop(N, E):
    EP = E // NSUB
    NCH = EP // C
    RP = N // NSUB
    NZ = RP // ZR
    mesh = plsc.VectorSubcoreMesh(core_axis_name="c", subcore_axis_name="s")
    out_t = (jax.ShapeDtypeStruct((2 * N, HH), jnp.float32),
             jax.ShapeDtypeStruct((2 * N, HH), jnp.float32))

    @functools.partial(
        pl.kernel, out_type=out_t, mesh=mesh,
        compiler_params=pltpu.CompilerParams(use_tc_tiling_on_sc=False),
        scratch_types=[
            pltpu.VMEM((C,), jnp.int32),
            pltpu.VMEM((C,), jnp.int32),
            pltpu.VMEM((C, HH), jnp.float32),
            pltpu.VMEM_SHARED((N, HH), jnp.float32),
            pltpu.SemaphoreType.DMA,
        ],
    )
    def prop(srca_h, dsta_h, srcb_h, dstb_h, ta_h, tb_h, z32_h,
             oa, ob, src_v, dst_v, rows_v, acc, sem):
        cid = lax.axis_index("c")
        sid = lax.axis_index("s")

        def run(src_h, dst_h, t_h, o):
            def zb(i, carry):
                pltpu.sync_copy(z32_h, acc.at[pl.ds(sid * RP + i * ZR, ZR)])
                return carry
            lax.fori_loop(0, NZ, zb, 0)
            plsc.subcore_barrier()

            def ch(j, carry):
                b = sid * EP + j * C
                pltpu.sync_copy(src_h.at[pl.ds(cid * E + b, C)], src_v)
                pltpu.sync_copy(dst_h.at[pl.ds(b, C)], dst_v)
                pltpu.async_copy(t_h.at[src_v], rows_v, sem).wait()
                pltpu.sync_copy(rows_v, acc.at[dst_v], add=True)
                return carry
            lax.fori_loop(0, NCH, ch, 0)
            plsc.subcore_barrier()

            rb = sid * RP
            pltpu.sync_copy(acc.at[pl.ds(rb, RP)],
                            o.at[pl.ds(cid * N + rb, RP)])

        run(srca_h, dsta_h, ta_h, oa)
        plsc.subcore_barrier()
        run(srcb_h, dstb_h, tb_h, ob)

    return prop


_ROWS = 3136


def _full(shape):
    return pl.BlockSpec(shape, lambda i: (0, 0))


def _rows(w):
    return pl.BlockSpec((_ROWS, w), lambda i: (i, 0))


def _halves():
    return pl.BlockSpec((2, _ROWS, HH), lambda i: (0, i, 0))


def _split_halves(v):
    return jnp.stack([v[:, :HH], v[:, HH:]])


def _tc_init(x, W1, b1r, degg, degs):
    N = x.shape[0]

    def body(x_ref, w1_ref, b1_ref, dg_ref, ds_ref,
             hid_ref, dig_ref, dis_ref, hsg_ref, hss_ref):
        h = x_ref[...] * w1_ref[...] + b1_ref[...]
        dig = lax.rsqrt(dg_ref[...] + 1.0)
        dis = lax.rsqrt(ds_ref[...] + 1.0)
        hid_ref[...] = h
        dig_ref[...] = dig
        dis_ref[...] = dis
        hsg_ref[...] = _split_halves(dig * h)
        hss_ref[...] = _split_halves(dis * h)

    f32 = jnp.float32
    out_shape = [jax.ShapeDtypeStruct((N, H), f32),
                 jax.ShapeDtypeStruct((N, 1), f32),
                 jax.ShapeDtypeStruct((N, 1), f32),
                 jax.ShapeDtypeStruct((2, N, HH), f32),
                 jax.ShapeDtypeStruct((2, N, HH), f32)]
    return pl.pallas_call(
        body, grid=(N // _ROWS,),
        in_specs=[_rows(1), _full((1, H)), _full((1, H)), _rows(1), _rows(1)],
        out_specs=[_rows(H), _rows(1), _rows(1), _halves(), _halves()],
        out_shape=out_shape,
    )(x, W1, b1r, degg, degs)


def _blend(h, dig, dis, pg, ps, wg, bg, ws, bs, ww, bw):
    pg2 = jnp.concatenate([pg[0], pg[1]], axis=1)
    ps2 = jnp.concatenate([ps[0], ps[1]], axis=1)
    og = dig * (pg2 + dig * h)
    os_ = dis * (ps2 + dis * h)
    xg = jax.nn.relu(jnp.dot(og, wg, preferred_element_type=jnp.float32) + bg)
    xs = jax.nn.relu(jnp.dot(os_, ws, preferred_element_type=jnp.float32) + bs)
    s = jax.nn.sigmoid(jnp.dot(h, ww, preferred_element_type=jnp.float32) + bw)
    return s * xg + (1.0 - s) * xs


def _tc_layer(hid, dig, dis, pg, ps, wg, bg, ws, bs, ww, bw):
    N = hid.shape[0]

    def body(h_ref, dig_ref, dis_ref, pg_ref, ps_ref,
             wg_ref, bg_ref, ws_ref, bs_ref, ww_ref, bw_ref,
             hid_ref, hsg_ref, hss_ref):
        dig = dig_ref[...]
        dis = dis_ref[...]
        hn = _blend(h_ref[...], dig, dis, pg_ref[...], ps_ref[...],
                    wg_ref[...], bg_ref[...], ws_ref[...], bs_ref[...],
                    ww_ref[...], bw_ref[...])
        hid_ref[...] = hn
        hsg_ref[...] = _split_halves(dig * hn)
        hss_ref[...] = _split_halves(dis * hn)

    f32 = jnp.float32
    out_shape = [jax.ShapeDtypeStruct((N, H), f32),
                 jax.ShapeDtypeStruct((2, N, HH), f32),
                 jax.ShapeDtypeStruct((2, N, HH), f32)]
    return pl.pallas_call(
        body, grid=(N // _ROWS,),
        in_specs=[_rows(H), _rows(1), _rows(1), _halves(), _halves(),
                  _full((H, H)), _full((1, H)), _full((H, H)), _full((1, H)),
                  _full((H, 1)), _full((1, 1))],
        out_specs=[_rows(H), _halves(), _halves()],
        out_shape=out_shape,
    )(hid, dig, dis, pg, ps, wg, bg, ws, bs, ww, bw)


def _tc_final(hid, dig, dis, pg, ps, batch2, wg, bg, ws, bs, ww, bw):
    N = hid.shape[0]

    def body(h_ref, dig_ref, dis_ref, pg_ref, ps_ref, b_ref,
             wg_ref, bg_ref, ws_ref, bs_ref, ww_ref, bw_ref, out_ref):
        hn = _blend(h_ref[...], dig_ref[...], dis_ref[...], pg_ref[...],
                    ps_ref[...], wg_ref[...], bg_ref[...], ws_ref[...],
                    bs_ref[...], ww_ref[...], bw_ref[...])
        bid = b_ref[...]
        neg = jnp.float32(-jnp.inf)
        m = jnp.stack([jnp.max(jnp.where(bid == g, hn, neg), axis=0)
                       for g in range(G)])

        @pl.when(pl.program_id(0) == 0)
        def _():
            out_ref[...] = jnp.full((G, H), neg, jnp.float32)

        out_ref[...] = jnp.maximum(out_ref[...], m)

    return pl.pallas_call(
        body, grid=(N // _ROWS,),
        in_specs=[_rows(H), _rows(1), _rows(1), _halves(), _halves(),
                  _rows(1),
                  _full((H, H)), _full((1, H)), _full((H, H)), _full((1, H)),
                  _full((H, 1)), _full((1, 1))],
        out_specs=pl.BlockSpec((G, H), lambda i: (0, 0)),
        out_shape=jax.ShapeDtypeStruct((G, H), jnp.float32),
    )(hid, dig, dis, pg, ps, batch2, wg, bg, ws, bs, ww, bw)


def kernel(x, edge_index, edge_index_sim, batch, W1, b1, Wg, bg, Ws, bs, Ww, bw):
    N = x.shape[0]
    E = edge_index.shape[1]
    L = Wg.shape[0]
    NP = -(-N // NPAD) * NPAD

    srcg = edge_index[0].astype(jnp.int32)
    dstg = edge_index[1].astype(jnp.int32)
    srcs = edge_index_sim[0].astype(jnp.int32)
    dsts = edge_index_sim[1].astype(jnp.int32)
    srcg2 = jnp.concatenate([srcg, srcg + NP])
    srcs2 = jnp.concatenate([srcs, srcs + NP])
    dstcat = jnp.concatenate([dstg, dsts])
    xp = jnp.pad(x, ((0, NP - N), (0, 0)))
    batch2 = jnp.pad(batch.reshape(N, 1).astype(jnp.int32),
                     ((0, NP - N), (0, 0)), constant_values=-1)
    b1r = b1.reshape(1, H)
    bgr = bg.reshape(L, 1, H)
    bsr = bs.reshape(L, 1, H)
    bwr = bw.reshape(L, 1, 1)

    ones_c = jnp.ones((C, 16), jnp.float32)
    z16 = jnp.zeros((ZR, 16), jnp.float32)
    z32 = jnp.zeros((ZR, HH), jnp.float32)

    degcat = _make_deg(NP, E)(dstcat, ones_c, z16)
    degg = degcat[:NP, :1]
    degs = degcat[NP:, :1]

    hid, dig, dis, hsg, hss = _tc_init(xp, W1, b1r, degg, degs)

    prop = _make_prop(NP, E)
    for i in range(L):
        prg, prs = prop(srcg2, dstg, srcs2, dsts,
                        hsg.reshape(2 * NP, HH), hss.reshape(2 * NP, HH), z32)
        pg = prg.reshape(2, NP, HH)
        ps = prs.reshape(2, NP, HH)
        if i < L - 1:
            hid, hsg, hss = _tc_layer(hid, dig, dis, pg, ps,
                                      Wg[i], bgr[i], Ws[i], bsr[i],
                                      Ww[i], bwr[i])
        else:
            out = _tc_final(hid, dig, dis, pg, ps, batch2,
                            Wg[i], bgr[i], Ws[i], bsr[i], Ww[i], bwr[i])
    return out
